# baseline (speedup 1.0000x reference)
import jax
import jax.numpy as jnp
from jax import lax
from jax.experimental import pallas as pl
from jax.experimental.pallas import tpu as pltpu
from jax.experimental.pallas import tpu_sc as plsc

N = 50000
E = 1600000
G = 64
NP = 50048
NROW = NP // 128
NSC = 2
NTILE = 16
TSLICE = NP // NTILE
ERW = 12672
RP = 6
RK = 11

_MESH = plsc.VectorSubcoreMesh(core_axis_name="c", subcore_axis_name="s")
_SC_PARAMS = pltpu.CompilerParams(use_tc_tiling_on_sc=False)
f32 = jnp.float32
i32 = jnp.int32


def _split(w, nworkers, nitems):
    q, r = nitems // nworkers, nitems % nworkers
    base = w * q + jnp.minimum(w, r)
    count = q + jnp.where(w < r, 1, 0)
    return base, count


def _k1_body(dst_h, batch_h, znp_h, z128_h, ones_h,
             deg_h, cnt_h,
             idxg, ones_v, stage, cbuf, deg_sp, cnt_sp, sem):
    c = lax.axis_index("c")
    s = lax.axis_index("s")
    w = s * NSC + c
    sync = pltpu.sync_copy
    sync(znp_h.at[pl.ds(s * TSLICE, TSLICE)], stage)
    sync(stage, deg_sp.at[pl.ds(s * TSLICE, TSLICE)])

    @pl.when(s == 0)
    def _():
        sync(z128_h, cbuf)
        sync(cbuf, cnt_sp)

    sync(ones_h, ones_v)
    plsc.subcore_barrier()

    wrows = ERW // (NSC * NTILE)
    nit = wrows // RK
    base = w * wrows
    sync(dst_h.at[pl.ds(base, RK)], idxg.at[0])

    def process(b, it):
        scps = [pltpu.async_copy(ones_v, deg_sp.at[idxg.at[b].at[j]], sem,
                                 add=True)
                for j in range(RK)]

        @pl.when(it + 1 < nit)
        def _():
            sync(dst_h.at[pl.ds(base + (it + 1) * RK, RK)], idxg.at[1 - b])

        for scp in scps:
            scp.wait()

    def pair(p, _):
        process(0, 2 * p)
        process(1, 2 * p + 1)
        return 0

    lax.fori_loop(0, nit // 2, pair, 0)

    rbase, rcount = _split(w, NSC * NTILE, NROW)

    def crow(r, _):
        sync(batch_h.at[rbase + r], idxg.at[0].at[pl.ds(0, 1)])
        sync(ones_v, cnt_sp.at[idxg.at[0].at[0]], add=True)
        return 0

    lax.fori_loop(0, rcount, crow, 0)

    plsc.subcore_barrier()

    sync(deg_sp.at[pl.ds(s * TSLICE, TSLICE)], stage)
    sync(stage, deg_h.at[c].at[pl.ds(s * TSLICE, TSLICE)])

    @pl.when(s == 0)
    def _():
        sync(cnt_sp, cbuf)
        sync(cbuf, cnt_h.at[c])


def _k1(dst3d, batch3d, znp, z128, ones128):
    return pl.kernel(
        _k1_body,
        out_type=(jax.ShapeDtypeStruct((NSC, NP), f32),
                  jax.ShapeDtypeStruct((NSC, 128), f32)),
        mesh=_MESH,
        compiler_params=_SC_PARAMS,
        scratch_types=[
            pltpu.VMEM((2, RK, 128), i32),
            pltpu.VMEM((128,), f32),
            pltpu.VMEM((TSLICE,), f32),
            pltpu.VMEM((128,), f32),
            pltpu.VMEM_SHARED((NP,), f32),
            pltpu.VMEM_SHARED((128,), f32),
            pltpu.SemaphoreType.DMA,
        ],
    )(dst3d, batch3d, znp, z128, ones128)


def _make_prop(edge_split):
    F = 32

    def body(src_h, dst_h, g_h, zf_h, acc_h,
             idxs0, idxd0, idxs1, idxd1, rows, acc_sp, sem, ssem):
        c = lax.axis_index("c")
        s = lax.axis_index("s")
        sync = pltpu.sync_copy
        sync(zf_h.at[pl.ds(s * TSLICE, TSLICE)],
             acc_sp.at[pl.ds(s * TSLICE, TSLICE)])
        plsc.subcore_barrier()

        if edge_split:
            w = s * NSC + c
            wrows = ERW // (NSC * NTILE)
            gc = g_h
        else:
            w = s
            wrows = ERW // NTILE
            gc = g_h.at[c]
        nit = wrows // RP
        base = w * wrows

        sync(src_h.at[pl.ds(base, RP)], idxs0)
        sync(dst_h.at[pl.ds(base, RP)], idxd0)

        def process(bs, bd, pf_s, pf_d, it):
            cps = [pltpu.async_copy(gc.at[bs.at[j]], rows.at[j], sem)
                   for j in range(RP)]

            @pl.when(it + 1 < nit)
            def _():
                sync(src_h.at[pl.ds(base + (it + 1) * RP, RP)], pf_s)
                sync(dst_h.at[pl.ds(base + (it + 1) * RP, RP)], pf_d)

            for cp in cps:
                cp.wait()
            scps = [pltpu.async_copy(rows.at[j], acc_sp.at[bd.at[j]], ssem,
                                     add=True)
                    for j in range(RP)]
            for scp in scps:
                scp.wait()

        def pair(p, _):
            process(idxs0, idxd0, idxs1, idxd1, 2 * p)
            process(idxs1, idxd1, idxs0, idxd0, 2 * p + 1)
            return 0

        lax.fori_loop(0, nit // 2, pair, 0)

        plsc.subcore_barrier()

        sync(acc_sp.at[pl.ds(s * TSLICE, TSLICE)],
             acc_h.at[c].at[pl.ds(s * TSLICE, TSLICE)])

    def run(src3d, dst3d, g, zf):
        return pl.kernel(
            body,
            out_type=jax.ShapeDtypeStruct((NSC, NP, F), f32),
            mesh=_MESH,
            compiler_params=_SC_PARAMS,
            scratch_types=[
                pltpu.VMEM((RP, 128), i32),
                pltpu.VMEM((RP, 128), i32),
                pltpu.VMEM((RP, 128), i32),
                pltpu.VMEM((RP, 128), i32),
                pltpu.VMEM((RP, 128, F), f32),
                pltpu.VMEM_SHARED((NP, F), f32),
                pltpu.SemaphoreType.DMA,
                pltpu.SemaphoreType.DMA,
            ],
        )(src3d, dst3d, g, zf)

    return run


_prop_edges = _make_prop(True)
_prop_feats = _make_prop(False)


def _k7_body(src_h, dst_h, g2_h, batch_h, dinv_h, znp_h, z128_h,
             pool_h,
             idxs, idxd, vals, dv, av, gv, bv, vv, pbuf, acc_sp, pool_sp,
             sem, sem2):
    c = lax.axis_index("c")
    s = lax.axis_index("s")
    w = s * NSC + c
    sync = pltpu.sync_copy
    sync(znp_h.at[pl.ds(s * TSLICE, TSLICE)], av.at[pl.ds(0, TSLICE)])
    sync(av.at[pl.ds(0, TSLICE)], acc_sp.at[pl.ds(s * TSLICE, TSLICE)])

    @pl.when(s == 0)
    def _():
        sync(z128_h, pbuf)
        sync(pbuf, pool_sp)

    plsc.subcore_barrier()

    wrows = ERW // (NSC * NTILE)
    nit = wrows // RK
    base = w * wrows
    sync(src_h.at[pl.ds(base, RK)], idxs.at[0])
    sync(dst_h.at[pl.ds(base, RK)], idxd.at[0])

    def process(b, it):
        cps = [pltpu.async_copy(g2_h.at[idxs.at[b].at[j]], vals.at[j], sem)
               for j in range(RK)]

        @pl.when(it + 1 < nit)
        def _():
            sync(src_h.at[pl.ds(base + (it + 1) * RK, RK)], idxs.at[1 - b])
            sync(dst_h.at[pl.ds(base + (it + 1) * RK, RK)], idxd.at[1 - b])

        for cp in cps:
            cp.wait()
        scps = [pltpu.async_copy(vals.at[j], acc_sp.at[idxd.at[b].at[j]],
                                 sem2, add=True)
                for j in range(RK)]
        for scp in scps:
            scp.wait()

    def pair(p, _):
        process(0, 2 * p)
        process(1, 2 * p + 1)
        return 0

    lax.fori_loop(0, nit // 2, pair, 0)

    plsc.subcore_barrier()

    q, r = NROW // NTILE, NROW % NTILE
    nb = s * q + jnp.minimum(s, r)
    sync(dinv_h.at[pl.ds(nb * 128, 24 * 128)], dv.at[pl.ds(0, 24 * 128)])
    sync(acc_sp.at[pl.ds(nb * 128, 24 * 128)], av.at[pl.ds(0, 24 * 128)])
    sync(g2_h.at[pl.ds(nb * 128, 24 * 128)], gv.at[pl.ds(0, 24 * 128)])
    sync(batch_h.at[pl.ds(nb, 24)], bv.at[pl.ds(0, 24)])

    @pl.when(s < r)
    def _():
        sync(dinv_h.at[pl.ds((nb + 24) * 128, 128)], dv.at[pl.ds(24 * 128, 128)])
        sync(acc_sp.at[pl.ds((nb + 24) * 128, 128)], av.at[pl.ds(24 * 128, 128)])
        sync(g2_h.at[pl.ds((nb + 24) * 128, 128)], gv.at[pl.ds(24 * 128, 128)])
        sync(batch_h.at[pl.ds(nb + 24, 1)], bv.at[pl.ds(24, 1)])

    nrows = 24 + jnp.where(s < r, 1, 0)

    def vrow(rr, _):
        for k in range(8):
            o = rr * 128 + k * 16
            d16 = dv[pl.ds(o, 16)]
            a16 = av[pl.ds(o, 16)]
            g16 = gv[pl.ds(o, 16)]
            zero = jnp.zeros((16,), f32)
            sv = jnp.where(c == 0, g16, zero)
            vv[rr, pl.ds(k * 16, 16)] = d16 * (a16 + sv)
        return 0

    lax.fori_loop(0, nrows, vrow, 0)

    def prow(rr, _):
        sync(vv.at[rr], pool_sp.at[bv.at[rr].at[0]], add=True)
        return 0

    lax.fori_loop(0, nrows, prow, 0)

    plsc.subcore_barrier()

    @pl.when(s == 0)
    def _():
        sync(pool_sp, pbuf)
        sync(pbuf, pool_h.at[c])


def _k7(src3d, dst3d, g2, batch3d, dinv, znp, z128):
    return pl.kernel(
        _k7_body,
        out_type=jax.ShapeDtypeStruct((NSC, 128), f32),
        mesh=_MESH,
        compiler_params=_SC_PARAMS,
        scratch_types=[
            pltpu.VMEM((2, RK, 128), i32),
            pltpu.VMEM((2, RK, 128), i32),
            pltpu.VMEM((RK, 128), f32),
            pltpu.VMEM((25 * 128,), f32),
            pltpu.VMEM((25 * 128,), f32),
            pltpu.VMEM((25 * 128,), f32),
            pltpu.VMEM((25, 1, 128), i32),
            pltpu.VMEM((25, 128), f32),
            pltpu.VMEM((128,), f32),
            pltpu.VMEM_SHARED((NP,), f32),
            pltpu.VMEM_SHARED((128,), f32),
            pltpu.SemaphoreType.DMA,
            pltpu.SemaphoreType.DMA,
        ],
    )(src3d, dst3d, g2, batch3d, dinv, znp, z128)


BR = 2176
GRID = NP // BR


def _k2_body(deg_ref, x_ref, dinv_ref, g0_ref):
    d = deg_ref[0] + deg_ref[1] + 1.0
    dinv = lax.rsqrt(d)
    dinv_ref[...] = dinv
    g0_ref[...] = dinv * x_ref[...]


def _k2(deg, xp):
    return pl.pallas_call(
        _k2_body,
        grid=(GRID,),
        in_specs=[
            pl.BlockSpec((NSC, BR, 1), lambda i: (0, i, 0)),
            pl.BlockSpec((BR, 32), lambda i: (i, 0)),
        ],
        out_specs=[
            pl.BlockSpec((BR, 1), lambda i: (i, 0)),
            pl.BlockSpec((BR, 32), lambda i: (i, 0)),
        ],
        out_shape=[jax.ShapeDtypeStruct((NP, 1), f32),
                   jax.ShapeDtypeStruct((NP, 32), f32)],
    )(deg.reshape(NSC, NP, 1), xp)


def _k4_body(acc_ref, g0_ref, dinv_ref, w1_ref, b1_ref, w2_ref,
             g1_ref):
    a = acc_ref[0] + acc_ref[1]
    dinv = dinv_ref[...]
    p0 = dinv * (a + g0_ref[...])
    h1 = jnp.maximum(jnp.dot(p0, w1_ref[...],
                             preferred_element_type=f32) + b1_ref[...], 0.0)
    t1 = jnp.dot(h1, w2_ref[...], preferred_element_type=f32)
    g1 = dinv * t1
    g1_ref[0] = g1[:, :32]
    g1_ref[1] = g1[:, 32:]


def _k4(acc, g0, dinv, W1p, b1, W2):
    return pl.pallas_call(
        _k4_body,
        grid=(GRID,),
        in_specs=[
            pl.BlockSpec((NSC, BR, 32), lambda i: (0, i, 0)),
            pl.BlockSpec((BR, 32), lambda i: (i, 0)),
            pl.BlockSpec((BR, 1), lambda i: (i, 0)),
            pl.BlockSpec((32, 128), lambda i: (0, 0)),
            pl.BlockSpec((1, 128), lambda i: (0, 0)),
            pl.BlockSpec((128, 64), lambda i: (0, 0)),
        ],
        out_specs=pl.BlockSpec((NSC, BR, 32), lambda i: (0, i, 0)),
        out_shape=jax.ShapeDtypeStruct((NSC, NP, 32), f32),
    )(acc, g0, dinv, W1p, b1.reshape(1, 128), W2)


def _k6_body(acc_ref, g1_ref, dinv_ref, b2_ref, w3_ref, wl_ref,
             g2_ref):
    a = jnp.concatenate([acc_ref[0], acc_ref[1]], axis=-1)
    g = jnp.concatenate([g1_ref[0], g1_ref[1]], axis=-1)
    dinv = dinv_ref[...]
    p1 = dinv * (a + g)
    h2 = jnp.maximum(p1 + b2_ref[...], 0.0)
    w3l = jnp.dot(w3_ref[...], wl_ref[...], preferred_element_type=f32)
    sc = jnp.dot(h2, w3l, preferred_element_type=f32)
    g2_ref[...] = dinv * sc


def _k6(acc, g1, dinv, b2, W3, Wl):
    return pl.pallas_call(
        _k6_body,
        grid=(GRID,),
        in_specs=[
            pl.BlockSpec((NSC, BR, 32), lambda i: (0, i, 0)),
            pl.BlockSpec((NSC, BR, 32), lambda i: (0, i, 0)),
            pl.BlockSpec((BR, 1), lambda i: (i, 0)),
            pl.BlockSpec((1, 64), lambda i: (0, 0)),
            pl.BlockSpec((64, 32), lambda i: (0, 0)),
            pl.BlockSpec((32, 1), lambda i: (0, 0)),
        ],
        out_specs=pl.BlockSpec((BR, 1), lambda i: (i, 0)),
        out_shape=jax.ShapeDtypeStruct((NP, 1), f32),
    )(acc, g1, dinv, b2.reshape(1, 64), W3, Wl)


def _k8_body(pool_ref, cnt_ref, y_ref, b3_ref, wl_ref, bl_ref,
             out_ref, loss_ref):
    pooled = pool_ref[0, :G] + pool_ref[1, :G]
    cnt = cnt_ref[0, :G] + cnt_ref[1, :G]
    c3 = jnp.dot(b3_ref[...], wl_ref[...], preferred_element_type=f32)
    logits = pooled / jnp.maximum(cnt, 1.0) + c3[0, 0] + bl_ref[0, 0]
    t = y_ref[...].astype(f32)
    bce = (jnp.maximum(logits, 0.0) - logits * t
           + jnp.log1p(jnp.exp(-jnp.abs(logits))))
    loss_ref[...] = jnp.sum(bce, axis=0, keepdims=True) / G
    out_ref[...] = 1.0 / (1.0 + jnp.exp(-logits))


def _k8(poolp, cntp, y, b3, Wl, bl):
    return pl.pallas_call(
        _k8_body,
        grid=(1,),
        in_specs=[
            pl.BlockSpec((NSC, 128, 1), lambda i: (0, 0, 0)),
            pl.BlockSpec((NSC, 128, 1), lambda i: (0, 0, 0)),
            pl.BlockSpec((G, 1), lambda i: (0, 0)),
            pl.BlockSpec((1, 32), lambda i: (0, 0)),
            pl.BlockSpec((32, 1), lambda i: (0, 0)),
            pl.BlockSpec((1, 1), lambda i: (0, 0)),
        ],
        out_specs=[pl.BlockSpec((G, 1), lambda i: (0, 0)),
                   pl.BlockSpec((1, 1), lambda i: (0, 0))],
        out_shape=[jax.ShapeDtypeStruct((G, 1), f32),
                   jax.ShapeDtypeStruct((1, 1), f32)],
    )(poolp, cntp, y.reshape(G, 1), b3.reshape(1, 32), Wl, bl.reshape(1, 1))


def kernel(x, y, edge_index, batch, W1, b1, W2, b2, W3, b3, Wl, bl):
    pad = jnp.full((2, ERW * 128 - E), NP - 1, dtype=edge_index.dtype)
    ei = jnp.concatenate([edge_index, pad], axis=1)
    src2d = ei[0].reshape(ERW, 128)
    dst2d = ei[1].reshape(ERW, 128)
    xp = jnp.pad(x, ((0, NP - N), (0, 2)))
    batchp = jnp.pad(batch, (0, NP - N), constant_values=G)
    batch3d = batchp.reshape(NROW, 1, 128)
    W1p = jnp.pad(W1, ((0, 2), (0, 0)))
    znp = jnp.zeros((NP,), f32)
    z128 = jnp.zeros((128,), f32)
    z32 = jnp.zeros((NP, 32), f32)

    ones128 = jnp.ones((128,), f32)
    deg, cnt = _k1(dst2d, batch3d, znp, z128, ones128)
    dinv, g0 = _k2(deg, xp)
    acc0 = _prop_edges(src2d, dst2d, g0, z32)
    g1 = _k4(acc0, g0, dinv, W1p, b1, W2)
    acc1 = _prop_feats(src2d, dst2d, g1, z32)
    g2 = _k6(acc1, g1, dinv, b2, W3, Wl)
    pool = _k7(src2d, dst2d, g2.reshape(NP), batch3d,
               dinv.reshape(NP), znp, z128)
    out, loss = _k8(pool.reshape(NSC, 128, 1), cnt.reshape(NSC, 128, 1),
                    y, b3, Wl, bl)
    return (out, loss.reshape(()))


# speedup vs baseline: 1.4024x; 1.4024x over previous
import jax
import jax.numpy as jnp
from jax import lax
from jax.experimental import pallas as pl
from jax.experimental.pallas import tpu as pltpu
from jax.experimental.pallas import tpu_sc as plsc

N = 50000
E = 1600000
G = 64
NP = 50048
NROW = NP // 128
ECH = E // 512
NSC = 2
NTILE = 16
TSLICE = NP // NTILE

_MESH = plsc.VectorSubcoreMesh(core_axis_name="c", subcore_axis_name="s")
_SC_PARAMS = pltpu.CompilerParams(use_tc_tiling_on_sc=False)
f32 = jnp.float32
i32 = jnp.int32


def _split(w, nworkers, nitems):
    q, r = nitems // nworkers, nitems % nworkers
    base = w * q + jnp.minimum(w, r)
    count = q + jnp.where(w < r, 1, 0)
    return base, count


def _k1_body(dst_h, batch_h, znp_h, z128_h, ones_h,
             deg_h, cnt_h,
             idxg, ones_v, stage, cbuf, deg_sp, cnt_sp, sem):
    c = lax.axis_index("c")
    s = lax.axis_index("s")
    w = s * NSC + c
    sync = pltpu.sync_copy
    sync(znp_h.at[pl.ds(s * TSLICE, TSLICE)], stage)
    sync(stage, deg_sp.at[pl.ds(s * TSLICE, TSLICE)])

    @pl.when(s == 0)
    def _():
        sync(z128_h, cbuf)
        sync(cbuf, cnt_sp)

    sync(ones_h, ones_v)
    plsc.subcore_barrier()

    base, count = _split(w, NSC * NTILE, ECH)
    sync(dst_h.at[base], idxg.at[0])

    def process(b, ch):
        scps = [pltpu.async_copy(ones_v, deg_sp.at[idxg.at[b].at[j]], sem,
                                 add=True)
                for j in range(4)]

        @pl.when(ch + 1 < count)
        def _():
            sync(dst_h.at[base + ch + 1], idxg.at[1 - b])

        for scp in scps:
            scp.wait()

    def pair(p, _):
        process(0, 2 * p)

        @pl.when(2 * p + 1 < count)
        def _():
            process(1, 2 * p + 1)

        return 0

    lax.fori_loop(0, (count + 1) // 2, pair, 0)

    rbase, rcount = _split(w, NSC * NTILE, NROW)

    def crow(r, _):
        sync(batch_h.at[rbase + r], idxg.at[0].at[pl.ds(0, 1)])
        sync(ones_v, cnt_sp.at[idxg.at[0].at[0]], add=True)
        return 0

    lax.fori_loop(0, rcount, crow, 0)

    plsc.subcore_barrier()

    sync(deg_sp.at[pl.ds(s * TSLICE, TSLICE)], stage)
    sync(stage, deg_h.at[c].at[pl.ds(s * TSLICE, TSLICE)])

    @pl.when(s == 0)
    def _():
        sync(cnt_sp, cbuf)
        sync(cbuf, cnt_h.at[c])


def _k1(dst3d, batch3d, znp, z128, ones128):
    return pl.kernel(
        _k1_body,
        out_type=(jax.ShapeDtypeStruct((NSC, NP), f32),
                  jax.ShapeDtypeStruct((NSC, 128), f32)),
        mesh=_MESH,
        compiler_params=_SC_PARAMS,
        scratch_types=[
            pltpu.VMEM((2, 4, 128), i32),
            pltpu.VMEM((128,), f32),
            pltpu.VMEM((TSLICE,), f32),
            pltpu.VMEM((128,), f32),
            pltpu.VMEM_SHARED((NP,), f32),
            pltpu.VMEM_SHARED((128,), f32),
            pltpu.SemaphoreType.DMA,
        ],
    )(dst3d, batch3d, znp, z128, ones128)


def _make_prop(edge_split):
    F = 32

    def body(src_h, dst_h, g_h, zf_h, acc_h,
             idxs0, idxd0, idxs1, idxd1, rows, acc_sp, sem, ssem):
        c = lax.axis_index("c")
        s = lax.axis_index("s")
        sync = pltpu.sync_copy
        sync(zf_h.at[pl.ds(s * TSLICE, TSLICE)],
             acc_sp.at[pl.ds(s * TSLICE, TSLICE)])
        plsc.subcore_barrier()

        if edge_split:
            w = s * NSC + c
            base, count = _split(w, NSC * NTILE, ECH)
            gc = g_h
        else:
            base, count = _split(s, NTILE, ECH)
            gc = g_h.at[c]

        sync(src_h.at[base], idxs0)
        sync(dst_h.at[base], idxd0)

        def process(bs, bd, pf_s, pf_d, ch):
            cps = [pltpu.async_copy(gc.at[bs.at[j]], rows.at[j], sem)
                   for j in range(4)]

            @pl.when(ch + 1 < count)
            def _():
                sync(src_h.at[base + ch + 1], pf_s)
                sync(dst_h.at[base + ch + 1], pf_d)

            for cp in cps:
                cp.wait()
            scps = [pltpu.async_copy(rows.at[j], acc_sp.at[bd.at[j]], ssem,
                                     add=True)
                    for j in range(4)]
            for scp in scps:
                scp.wait()

        def pair(p, _):
            process(idxs0, idxd0, idxs1, idxd1, 2 * p)

            @pl.when(2 * p + 1 < count)
            def _():
                process(idxs1, idxd1, idxs0, idxd0, 2 * p + 1)

            return 0

        lax.fori_loop(0, (count + 1) // 2, pair, 0)

        plsc.subcore_barrier()

        sync(acc_sp.at[pl.ds(s * TSLICE, TSLICE)],
             acc_h.at[c].at[pl.ds(s * TSLICE, TSLICE)])

    def run(src3d, dst3d, g, zf):
        return pl.kernel(
            body,
            out_type=jax.ShapeDtypeStruct((NSC, NP, F), f32),
            mesh=_MESH,
            compiler_params=_SC_PARAMS,
            scratch_types=[
                pltpu.VMEM((4, 128), i32),
                pltpu.VMEM((4, 128), i32),
                pltpu.VMEM((4, 128), i32),
                pltpu.VMEM((4, 128), i32),
                pltpu.VMEM((4, 128, F), f32),
                pltpu.VMEM_SHARED((NP, F), f32),
                pltpu.SemaphoreType.DMA,
                pltpu.SemaphoreType.DMA,
            ],
        )(src3d, dst3d, g, zf)

    return run


_prop_edges = _make_prop(True)
_prop_feats = _make_prop(False)


def _k7_body(src_h, dst_h, g2_h, batch_h, dinv_h, znp_h, z128_h,
             pool_h,
             idxs, idxd, vals, dv, av, gv, bv, vv, pbuf, acc_sp, pool_sp,
             sem, sem2):
    c = lax.axis_index("c")
    s = lax.axis_index("s")
    w = s * NSC + c
    sync = pltpu.sync_copy
    sync(znp_h.at[pl.ds(s * TSLICE, TSLICE)], av.at[pl.ds(0, TSLICE)])
    sync(av.at[pl.ds(0, TSLICE)], acc_sp.at[pl.ds(s * TSLICE, TSLICE)])

    @pl.when(s == 0)
    def _():
        sync(z128_h, pbuf)
        sync(pbuf, pool_sp)

    plsc.subcore_barrier()

    base, count = _split(w, NSC * NTILE, ECH)
    sync(src_h.at[base], idxs.at[0])
    sync(dst_h.at[base], idxd.at[0])

    def process(b, ch):
        cps = [pltpu.async_copy(g2_h.at[idxs.at[b].at[j]], vals.at[j], sem)
               for j in range(4)]

        @pl.when(ch + 1 < count)
        def _():
            sync(src_h.at[base + ch + 1], idxs.at[1 - b])
            sync(dst_h.at[base + ch + 1], idxd.at[1 - b])

        for cp in cps:
            cp.wait()
        scps = [pltpu.async_copy(vals.at[j], acc_sp.at[idxd.at[b].at[j]],
                                 sem2, add=True)
                for j in range(4)]
        for scp in scps:
            scp.wait()

    def pair(p, _):
        process(0, 2 * p)

        @pl.when(2 * p + 1 < count)
        def _():
            process(1, 2 * p + 1)

        return 0

    lax.fori_loop(0, (count + 1) // 2, pair, 0)

    plsc.subcore_barrier()

    q, r = NROW // NTILE, NROW % NTILE
    nb = s * q + jnp.minimum(s, r)
    sync(dinv_h.at[pl.ds(nb * 128, 24 * 128)], dv.at[pl.ds(0, 24 * 128)])
    sync(acc_sp.at[pl.ds(nb * 128, 24 * 128)], av.at[pl.ds(0, 24 * 128)])
    sync(g2_h.at[pl.ds(nb * 128, 24 * 128)], gv.at[pl.ds(0, 24 * 128)])
    sync(batch_h.at[pl.ds(nb, 24)], bv.at[pl.ds(0, 24)])

    @pl.when(s < r)
    def _():
        sync(dinv_h.at[pl.ds((nb + 24) * 128, 128)], dv.at[pl.ds(24 * 128, 128)])
        sync(acc_sp.at[pl.ds((nb + 24) * 128, 128)], av.at[pl.ds(24 * 128, 128)])
        sync(g2_h.at[pl.ds((nb + 24) * 128, 128)], gv.at[pl.ds(24 * 128, 128)])
        sync(batch_h.at[pl.ds(nb + 24, 1)], bv.at[pl.ds(24, 1)])

    nrows = 24 + jnp.where(s < r, 1, 0)

    def vrow(rr, _):
        for k in range(8):
            o = rr * 128 + k * 16
            d16 = dv[pl.ds(o, 16)]
            a16 = av[pl.ds(o, 16)]
            g16 = gv[pl.ds(o, 16)]
            zero = jnp.zeros((16,), f32)
            sv = jnp.where(c == 0, g16, zero)
            vv[rr, pl.ds(k * 16, 16)] = d16 * (a16 + sv)
        return 0

    lax.fori_loop(0, nrows, vrow, 0)

    def prow(rr, _):
        sync(vv.at[rr], pool_sp.at[bv.at[rr].at[0]], add=True)
        return 0

    lax.fori_loop(0, nrows, prow, 0)

    plsc.subcore_barrier()

    @pl.when(s == 0)
    def _():
        sync(pool_sp, pbuf)
        sync(pbuf, pool_h.at[c])


def _k7(src3d, dst3d, g2, batch3d, dinv, znp, z128):
    return pl.kernel(
        _k7_body,
        out_type=jax.ShapeDtypeStruct((NSC, 128), f32),
        mesh=_MESH,
        compiler_params=_SC_PARAMS,
        scratch_types=[
            pltpu.VMEM((2, 4, 128), i32),
            pltpu.VMEM((2, 4, 128), i32),
            pltpu.VMEM((4, 128), f32),
            pltpu.VMEM((25 * 128,), f32),
            pltpu.VMEM((25 * 128,), f32),
            pltpu.VMEM((25 * 128,), f32),
            pltpu.VMEM((25, 1, 128), i32),
            pltpu.VMEM((25, 128), f32),
            pltpu.VMEM((128,), f32),
            pltpu.VMEM_SHARED((NP,), f32),
            pltpu.VMEM_SHARED((128,), f32),
            pltpu.SemaphoreType.DMA,
            pltpu.SemaphoreType.DMA,
        ],
    )(src3d, dst3d, g2, batch3d, dinv, znp, z128)


BR = 2176
GRID = NP // BR


def _k2_body(deg_ref, x_ref, dinv_ref, g0_ref):
    d = deg_ref[0] + deg_ref[1] + 1.0
    dinv = lax.rsqrt(d)
    dinv_ref[...] = dinv
    g0_ref[...] = dinv * x_ref[...]


def _k2(deg, xp):
    return pl.pallas_call(
        _k2_body,
        grid=(GRID,),
        in_specs=[
            pl.BlockSpec((NSC, BR, 1), lambda i: (0, i, 0)),
            pl.BlockSpec((BR, 32), lambda i: (i, 0)),
        ],
        out_specs=[
            pl.BlockSpec((BR, 1), lambda i: (i, 0)),
            pl.BlockSpec((BR, 32), lambda i: (i, 0)),
        ],
        out_shape=[jax.ShapeDtypeStruct((NP, 1), f32),
                   jax.ShapeDtypeStruct((NP, 32), f32)],
    )(deg.reshape(NSC, NP, 1), xp)


def _k4_body(acc_ref, g0_ref, dinv_ref, w1_ref, b1_ref, w2_ref,
             g1_ref):
    a = acc_ref[0] + acc_ref[1]
    dinv = dinv_ref[...]
    p0 = dinv * (a + g0_ref[...])
    h1 = jnp.maximum(jnp.dot(p0, w1_ref[...],
                             preferred_element_type=f32) + b1_ref[...], 0.0)
    t1 = jnp.dot(h1, w2_ref[...], preferred_element_type=f32)
    g1 = dinv * t1
    g1_ref[0] = g1[:, :32]
    g1_ref[1] = g1[:, 32:]


def _k4(acc, g0, dinv, W1p, b1, W2):
    return pl.pallas_call(
        _k4_body,
        grid=(GRID,),
        in_specs=[
            pl.BlockSpec((NSC, BR, 32), lambda i: (0, i, 0)),
            pl.BlockSpec((BR, 32), lambda i: (i, 0)),
            pl.BlockSpec((BR, 1), lambda i: (i, 0)),
            pl.BlockSpec((32, 128), lambda i: (0, 0)),
            pl.BlockSpec((1, 128), lambda i: (0, 0)),
            pl.BlockSpec((128, 64), lambda i: (0, 0)),
        ],
        out_specs=pl.BlockSpec((NSC, BR, 32), lambda i: (0, i, 0)),
        out_shape=jax.ShapeDtypeStruct((NSC, NP, 32), f32),
    )(acc, g0, dinv, W1p, b1.reshape(1, 128), W2)


def _k6_body(acc_ref, g1_ref, dinv_ref, b2_ref, w3_ref, wl_ref,
             g2_ref):
    a = jnp.concatenate([acc_ref[0], acc_ref[1]], axis=-1)
    g = jnp.concatenate([g1_ref[0], g1_ref[1]], axis=-1)
    dinv = dinv_ref[...]
    p1 = dinv * (a + g)
    h2 = jnp.maximum(p1 + b2_ref[...], 0.0)
    w3l = jnp.dot(w3_ref[...], wl_ref[...], preferred_element_type=f32)
    sc = jnp.dot(h2, w3l, preferred_element_type=f32)
    g2_ref[...] = dinv * sc


def _k6(acc, g1, dinv, b2, W3, Wl):
    return pl.pallas_call(
        _k6_body,
        grid=(GRID,),
        in_specs=[
            pl.BlockSpec((NSC, BR, 32), lambda i: (0, i, 0)),
            pl.BlockSpec((NSC, BR, 32), lambda i: (0, i, 0)),
            pl.BlockSpec((BR, 1), lambda i: (i, 0)),
            pl.BlockSpec((1, 64), lambda i: (0, 0)),
            pl.BlockSpec((64, 32), lambda i: (0, 0)),
            pl.BlockSpec((32, 1), lambda i: (0, 0)),
        ],
        out_specs=pl.BlockSpec((BR, 1), lambda i: (i, 0)),
        out_shape=jax.ShapeDtypeStruct((NP, 1), f32),
    )(acc, g1, dinv, b2.reshape(1, 64), W3, Wl)


def _k8_body(pool_ref, cnt_ref, y_ref, b3_ref, wl_ref, bl_ref,
             out_ref, loss_ref):
    pooled = pool_ref[0, :G] + pool_ref[1, :G]
    cnt = cnt_ref[0, :G] + cnt_ref[1, :G]
    c3 = jnp.dot(b3_ref[...], wl_ref[...], preferred_element_type=f32)
    logits = pooled / jnp.maximum(cnt, 1.0) + c3[0, 0] + bl_ref[0, 0]
    t = y_ref[...].astype(f32)
    bce = (jnp.maximum(logits, 0.0) - logits * t
           + jnp.log1p(jnp.exp(-jnp.abs(logits))))
    loss_ref[...] = jnp.sum(bce, axis=0, keepdims=True) / G
    out_ref[...] = 1.0 / (1.0 + jnp.exp(-logits))


def _k8(poolp, cntp, y, b3, Wl, bl):
    return pl.pallas_call(
        _k8_body,
        grid=(1,),
        in_specs=[
            pl.BlockSpec((NSC, 128, 1), lambda i: (0, 0, 0)),
            pl.BlockSpec((NSC, 128, 1), lambda i: (0, 0, 0)),
            pl.BlockSpec((G, 1), lambda i: (0, 0)),
            pl.BlockSpec((1, 32), lambda i: (0, 0)),
            pl.BlockSpec((32, 1), lambda i: (0, 0)),
            pl.BlockSpec((1, 1), lambda i: (0, 0)),
        ],
        out_specs=[pl.BlockSpec((G, 1), lambda i: (0, 0)),
                   pl.BlockSpec((1, 1), lambda i: (0, 0))],
        out_shape=[jax.ShapeDtypeStruct((G, 1), f32),
                   jax.ShapeDtypeStruct((1, 1), f32)],
    )(poolp, cntp, y.reshape(G, 1), b3.reshape(1, 32), Wl, bl.reshape(1, 1))


def kernel(x, y, edge_index, batch, W1, b1, W2, b2, W3, b3, Wl, bl):
    src3d = edge_index[0].reshape(ECH, 4, 128)
    dst3d = edge_index[1].reshape(ECH, 4, 128)
    xp = jnp.pad(x, ((0, NP - N), (0, 2)))
    batchp = jnp.pad(batch, (0, NP - N), constant_values=G)
    batch3d = batchp.reshape(NROW, 1, 128)
    W1p = jnp.pad(W1, ((0, 2), (0, 0)))
    znp = jnp.zeros((NP,), f32)
    z128 = jnp.zeros((128,), f32)
    z32 = jnp.zeros((NP, 32), f32)

    ones128 = jnp.ones((128,), f32)
    deg, cnt = _k1(dst3d, batch3d, znp, z128, ones128)
    dinv, g0 = _k2(deg, xp)
    acc0 = _prop_edges(src3d, dst3d, g0, z32)
    g1 = _k4(acc0, g0, dinv, W1p, b1, W2)
    acc1 = _prop_feats(src3d, dst3d, g1, z32)
    g2 = _k6(acc1, g1, dinv, b2, W3, Wl)
    pool = _k7(src3d, dst3d, g2.reshape(NP), batch3d,
               dinv.reshape(NP), znp, z128)
    out, loss = _k8(pool.reshape(NSC, 128, 1), cnt.reshape(NSC, 128, 1),
                    y, b3, Wl, bl)
    return (out, loss.reshape(()))
